# baseline (speedup 1.0000x reference)
import jax
import jax.numpy as jnp
from jax.experimental import pallas as pl
from jax.experimental.pallas import tpu as pltpu

_E = 8
_D = 1024
_DFF = 4096
_TM = 512
_CF = 512
_NCK = _DFF // _CF


def _router_body(hs_ref, rw_ref, scores_ref, xs_ref):
    tt = pl.program_id(0)
    x = hs_ref[...]
    logits = jax.lax.dot_general(
        x, rw_ref[...], (((1,), (1,)), ((), ())),
        preferred_element_type=jnp.float32)
    amax = jnp.argmax(logits, axis=1)
    mval = jnp.max(logits, axis=1)
    sig = jax.nn.sigmoid(mval)
    eid = jax.lax.broadcasted_iota(jnp.int32, (_E, _TM), 0)
    scores_ref[...] = jnp.where(eid == amax[None, :], sig[None, :], 0.0)

    @pl.when(tt == 0)
    def _():
        s8 = jnp.where(amax[:_E] == 0, sig[:_E], 0.0)
        xs_ref[...] = (x[:_E, :] * s8[:, None]).reshape(_E, 1, _D)


def _experts_body(slots_ref, nfir_ref, xs_ref, wg_ref, wu_ref, wd_ref, v_ref):
    k = pl.program_id(0)
    c = pl.program_id(1)

    @pl.when(jnp.logical_and(k == 0, c == 0))
    def _():
        v_ref[...] = jnp.zeros_like(v_ref)

    @pl.when(k < nfir_ref[0])
    def _():
        x = xs_ref[0]
        g = jax.lax.dot_general(x, wg_ref[0], (((1,), (1,)), ((), ())),
                                preferred_element_type=jnp.float32)
        u = jax.lax.dot_general(x, wu_ref[0], (((1,), (1,)), ((), ())),
                                preferred_element_type=jnp.float32)
        a = g * jax.nn.sigmoid(g) * u
        pv = jax.lax.dot_general(a, wd_ref[0], (((1,), (1,)), ((), ())),
                                 preferred_element_type=jnp.float32)
        v_ref[0:1, :] += pv


def _we_chunk(k, c, slots, nfir):
    return jnp.where(k >= nfir[0], _NCK - 1, c)


def _wg_im(k, c, slots, nfir):
    return (slots[k], _we_chunk(k, c, slots, nfir), 0)


def _wd_im(k, c, slots, nfir):
    return (slots[k], 0, _we_chunk(k, c, slots, nfir))


def _xs_im(k, c, slots, nfir):
    return (slots[k], 0, 0)


def _shared_body(hs_ref, wg_ref, wu_ref, wd_ref, v_ref, out_ref):
    fc = pl.program_id(1)

    @pl.when(fc == 0)
    def _():
        out_ref[...] = jnp.broadcast_to(v_ref[0:1, :], out_ref.shape)

    x = hs_ref[...].astype(jnp.bfloat16)
    g = jax.lax.dot_general(x, wg_ref[...].astype(jnp.bfloat16),
                            (((1,), (1,)), ((), ())),
                            preferred_element_type=jnp.float32)
    u = jax.lax.dot_general(x, wu_ref[...].astype(jnp.bfloat16),
                            (((1,), (1,)), ((), ())),
                            preferred_element_type=jnp.float32)
    a = (g * jax.nn.sigmoid(g) * u).astype(jnp.bfloat16)
    out_ref[...] += jax.lax.dot_general(
        a, wd_ref[...].astype(jnp.bfloat16), (((1,), (1,)), ((), ())),
        preferred_element_type=jnp.float32)


def kernel(hidden_states, router_w, shared_wg, shared_wu, shared_wd,
           exp_wg, exp_wu, exp_wd):
    b, s, d = hidden_states.shape
    hs = hidden_states.reshape(-1, d)
    t = hs.shape[0]

    scores, xs = pl.pallas_call(
        _router_body,
        grid=(t // _TM,),
        in_specs=[
            pl.BlockSpec((_TM, _D), lambda i: (i, 0)),
            pl.BlockSpec((_E, _D), lambda i: (0, 0)),
        ],
        out_specs=[
            pl.BlockSpec((_E, _TM), lambda i: (0, i)),
            pl.BlockSpec((_E, 1, _D), lambda i: (0, 0, 0)),
        ],
        out_shape=[
            jax.ShapeDtypeStruct((_E, t), jnp.float32),
            jax.ShapeDtypeStruct((_E, 1, _D), jnp.float32),
        ],
    )(hs, router_w)

    col = scores[0, :_E]
    mask = col != 0.0
    nfir = jnp.sum(mask.astype(jnp.int32)).reshape(1)
    order = jnp.argsort(jnp.logical_not(mask), stable=True).astype(jnp.int32)
    slots = order[jnp.clip(jnp.arange(_E), 0, jnp.maximum(nfir[0] - 1, 0))]

    v8 = pl.pallas_call(
        _experts_body,
        grid_spec=pltpu.PrefetchScalarGridSpec(
            num_scalar_prefetch=2,
            grid=(_E, _NCK),
            in_specs=[
                pl.BlockSpec((1, 1, _D), _xs_im),
                pl.BlockSpec((1, _CF, _D), _wg_im),
                pl.BlockSpec((1, _CF, _D), _wg_im),
                pl.BlockSpec((1, _D, _CF), _wd_im),
            ],
            out_specs=pl.BlockSpec((_E, _D), lambda k, c, slots, nfir: (0, 0)),
        ),
        out_shape=jax.ShapeDtypeStruct((_E, _D), jnp.float32),
        compiler_params=pltpu.CompilerParams(
            dimension_semantics=("arbitrary", "arbitrary")),
    )(slots, nfir, xs, exp_wg, exp_wu, exp_wd)

    out = pl.pallas_call(
        _shared_body,
        grid=(t // _TM, _NCK),
        in_specs=[
            pl.BlockSpec((_TM, _D), lambda i, f: (i, 0)),
            pl.BlockSpec((_CF, _D), lambda i, f: (f, 0)),
            pl.BlockSpec((_CF, _D), lambda i, f: (f, 0)),
            pl.BlockSpec((_D, _CF), lambda i, f: (0, f)),
            pl.BlockSpec((_E, _D), lambda i, f: (0, 0)),
        ],
        out_specs=pl.BlockSpec((_TM, _D), lambda i, f: (i, 0)),
        out_shape=jax.ShapeDtypeStruct((t, _D), jnp.float32),
        compiler_params=pltpu.CompilerParams(
            dimension_semantics=("parallel", "arbitrary")),
    )(hs, shared_wg, shared_wu, shared_wd, v8)

    return out, scores


# speedup vs baseline: 1.2480x; 1.2480x over previous
import jax
import jax.numpy as jnp
from jax.experimental import pallas as pl
from jax.experimental.pallas import tpu as pltpu

_E = 8
_D = 1024
_DFF = 4096
_TM = 512
_TMB = 2048
_CF = 512
_NCK = _DFF // _CF


def _router_body(hs_ref, rw_ref, scores_ref, xs_ref):
    tt = pl.program_id(0)
    x = hs_ref[...]
    logits = jax.lax.dot_general(
        x, rw_ref[...], (((1,), (1,)), ((), ())),
        preferred_element_type=jnp.float32)
    amax = jnp.argmax(logits, axis=1)
    mval = jnp.max(logits, axis=1)
    sig = jax.nn.sigmoid(mval)
    eid = jax.lax.broadcasted_iota(jnp.int32, (_E, _TM), 0)
    scores_ref[...] = jnp.where(eid == amax[None, :], sig[None, :], 0.0)

    @pl.when(tt == 0)
    def _():
        s8 = jnp.where(amax[:_E] == 0, sig[:_E], 0.0)
        xs_ref[...] = (x[:_E, :] * s8[:, None]).reshape(_E, 1, _D)


def _experts_body(slots_ref, nfir_ref, xs_ref, wg_ref, wu_ref, wd_ref, v_ref):
    k = pl.program_id(0)
    c = pl.program_id(1)

    @pl.when(jnp.logical_and(k == 0, c == 0))
    def _():
        v_ref[...] = jnp.zeros_like(v_ref)

    @pl.when(k < nfir_ref[0])
    def _():
        x = xs_ref[0]
        g = jax.lax.dot_general(x, wg_ref[0], (((1,), (1,)), ((), ())),
                                preferred_element_type=jnp.float32)
        u = jax.lax.dot_general(x, wu_ref[0], (((1,), (1,)), ((), ())),
                                preferred_element_type=jnp.float32)
        a = g * jax.nn.sigmoid(g) * u
        pv = jax.lax.dot_general(a, wd_ref[0], (((1,), (1,)), ((), ())),
                                 preferred_element_type=jnp.float32)
        v_ref[0:1, :] += pv


def _we_chunk(k, c, slots, nfir):
    return jnp.where(k >= nfir[0], _NCK - 1, c)


def _wg_im(k, c, slots, nfir):
    return (slots[k], _we_chunk(k, c, slots, nfir), 0)


def _wd_im(k, c, slots, nfir):
    return (slots[k], 0, _we_chunk(k, c, slots, nfir))


def _xs_im(k, c, slots, nfir):
    return (slots[k], 0, 0)


def _shared_body(hs_ref, wg_ref, wu_ref, wd_ref, v_ref, out_ref):
    fc = pl.program_id(1)

    @pl.when(fc == 0)
    def _():
        out_ref[...] = jnp.broadcast_to(v_ref[0:1, :], out_ref.shape)

    x = hs_ref[...].astype(jnp.bfloat16)
    g = jax.lax.dot_general(x, wg_ref[...].astype(jnp.bfloat16),
                            (((1,), (1,)), ((), ())),
                            preferred_element_type=jnp.float32)
    u = jax.lax.dot_general(x, wu_ref[...].astype(jnp.bfloat16),
                            (((1,), (1,)), ((), ())),
                            preferred_element_type=jnp.float32)
    a = (g * jax.nn.sigmoid(g) * u).astype(jnp.bfloat16)
    out_ref[...] += jax.lax.dot_general(
        a, wd_ref[...].astype(jnp.bfloat16), (((1,), (1,)), ((), ())),
        preferred_element_type=jnp.float32)


def kernel(hidden_states, router_w, shared_wg, shared_wu, shared_wd,
           exp_wg, exp_wu, exp_wd):
    b, s, d = hidden_states.shape
    hs = hidden_states.reshape(-1, d)
    t = hs.shape[0]

    scores, xs = pl.pallas_call(
        _router_body,
        grid=(t // _TM,),
        in_specs=[
            pl.BlockSpec((_TM, _D), lambda i: (i, 0)),
            pl.BlockSpec((_E, _D), lambda i: (0, 0)),
        ],
        out_specs=[
            pl.BlockSpec((_E, _TM), lambda i: (0, i)),
            pl.BlockSpec((_E, 1, _D), lambda i: (0, 0, 0)),
        ],
        out_shape=[
            jax.ShapeDtypeStruct((_E, t), jnp.float32),
            jax.ShapeDtypeStruct((_E, 1, _D), jnp.float32),
        ],
    )(hs, router_w)

    col = scores[0, :_E]
    mask = col != 0.0
    nfir = jnp.sum(mask.astype(jnp.int32)).reshape(1)
    order = jnp.argsort(jnp.logical_not(mask), stable=True).astype(jnp.int32)
    slots = order[jnp.clip(jnp.arange(_E), 0, jnp.maximum(nfir[0] - 1, 0))]

    v8 = pl.pallas_call(
        _experts_body,
        grid_spec=pltpu.PrefetchScalarGridSpec(
            num_scalar_prefetch=2,
            grid=(_E, _NCK),
            in_specs=[
                pl.BlockSpec((1, 1, _D), _xs_im),
                pl.BlockSpec((1, _CF, _D), _wg_im),
                pl.BlockSpec((1, _CF, _D), _wg_im),
                pl.BlockSpec((1, _D, _CF), _wd_im),
            ],
            out_specs=pl.BlockSpec((_E, _D), lambda k, c, slots, nfir: (0, 0)),
        ),
        out_shape=jax.ShapeDtypeStruct((_E, _D), jnp.float32),
        compiler_params=pltpu.CompilerParams(
            dimension_semantics=("arbitrary", "arbitrary")),
    )(slots, nfir, xs, exp_wg, exp_wu, exp_wd)

    out = pl.pallas_call(
        _shared_body,
        grid=(t // _TMB, _NCK),
        in_specs=[
            pl.BlockSpec((_TMB, _D), lambda i, f: (i, 0)),
            pl.BlockSpec((_CF, _D), lambda i, f: (f, 0)),
            pl.BlockSpec((_CF, _D), lambda i, f: (f, 0)),
            pl.BlockSpec((_D, _CF), lambda i, f: (0, f)),
            pl.BlockSpec((_E, _D), lambda i, f: (0, 0)),
        ],
        out_specs=pl.BlockSpec((_TMB, _D), lambda i, f: (i, 0)),
        out_shape=jax.ShapeDtypeStruct((t, _D), jnp.float32),
        compiler_params=pltpu.CompilerParams(
            dimension_semantics=("parallel", "arbitrary")),
    )(hs, shared_wg, shared_wu, shared_wd, v8)

    return out, scores


# speedup vs baseline: 1.2845x; 1.0293x over previous
import jax
import jax.numpy as jnp
from jax.experimental import pallas as pl
from jax.experimental.pallas import tpu as pltpu

_E = 8
_D = 1024
_DFF = 4096
_TM = 512
_TMB = 2048
_CF = 512
_NCK = _DFF // _CF


def _router_body(hs_ref, rw_ref, scores_ref, xs_ref):
    tt = pl.program_id(0)
    x = hs_ref[...]
    logits = jax.lax.dot_general(
        x, rw_ref[...], (((1,), (1,)), ((), ())),
        preferred_element_type=jnp.float32)
    amax = jnp.argmax(logits, axis=1)
    mval = jnp.max(logits, axis=1)
    sig = jax.nn.sigmoid(mval)
    eid = jax.lax.broadcasted_iota(jnp.int32, (_E, _TM), 0)
    scores_ref[...] = jnp.where(eid == amax[None, :], sig[None, :], 0.0)

    @pl.when(tt == 0)
    def _():
        s8 = jnp.where(amax[:_E] == 0, sig[:_E], 0.0)
        xs_ref[...] = (x[:_E, :] * s8[:, None]).reshape(_E, 1, _D)


def _experts_body(slots_ref, nfir_ref, xs_ref, wg_ref, wu_ref, wd_ref, v_ref):
    k = pl.program_id(0)
    c = pl.program_id(1)

    @pl.when(jnp.logical_and(k == 0, c == 0))
    def _():
        v_ref[...] = jnp.zeros_like(v_ref)

    @pl.when(k < nfir_ref[0])
    def _():
        x = xs_ref[0]
        g = jax.lax.dot_general(x, wg_ref[0], (((1,), (1,)), ((), ())),
                                preferred_element_type=jnp.float32)
        u = jax.lax.dot_general(x, wu_ref[0], (((1,), (1,)), ((), ())),
                                preferred_element_type=jnp.float32)
        a = g * jax.nn.sigmoid(g) * u
        pv = jax.lax.dot_general(a, wd_ref[0], (((1,), (1,)), ((), ())),
                                 preferred_element_type=jnp.float32)
        v_ref[0:1, :] += pv


def _we_chunk(k, c, slots, nfir):
    return jnp.where(k >= nfir[0], _NCK - 1, c)


def _wg_im(k, c, slots, nfir):
    return (slots[k], _we_chunk(k, c, slots, nfir), 0)


def _wd_im(k, c, slots, nfir):
    return (slots[k], 0, _we_chunk(k, c, slots, nfir))


def _xs_im(k, c, slots, nfir):
    return (slots[k], 0, 0)


def _shared_body(hs_ref, wg_ref, wu_ref, wd_ref, v_ref, out_ref):
    fc = pl.program_id(1)

    @pl.when(fc == 0)
    def _():
        out_ref[...] = jnp.broadcast_to(v_ref[0:1, :], out_ref.shape)

    x = hs_ref[...].astype(jnp.bfloat16)
    h = _CF // 2
    parts = []
    for j in range(2):
        wg = wg_ref[pl.ds(j * h, h), :].astype(jnp.bfloat16)
        wu = wu_ref[pl.ds(j * h, h), :].astype(jnp.bfloat16)
        wd = wd_ref[:, pl.ds(j * h, h)].astype(jnp.bfloat16)
        g = jax.lax.dot_general(x, wg, (((1,), (1,)), ((), ())),
                                preferred_element_type=jnp.float32)
        u = jax.lax.dot_general(x, wu, (((1,), (1,)), ((), ())),
                                preferred_element_type=jnp.float32)
        a = (g * jax.nn.sigmoid(g) * u).astype(jnp.bfloat16)
        parts.append(jax.lax.dot_general(
            a, wd, (((1,), (1,)), ((), ())),
            preferred_element_type=jnp.float32))
    out_ref[...] += parts[0] + parts[1]


def kernel(hidden_states, router_w, shared_wg, shared_wu, shared_wd,
           exp_wg, exp_wu, exp_wd):
    b, s, d = hidden_states.shape
    hs = hidden_states.reshape(-1, d)
    t = hs.shape[0]

    scores, xs = pl.pallas_call(
        _router_body,
        grid=(t // _TM,),
        in_specs=[
            pl.BlockSpec((_TM, _D), lambda i: (i, 0)),
            pl.BlockSpec((_E, _D), lambda i: (0, 0)),
        ],
        out_specs=[
            pl.BlockSpec((_E, _TM), lambda i: (0, i)),
            pl.BlockSpec((_E, 1, _D), lambda i: (0, 0, 0)),
        ],
        out_shape=[
            jax.ShapeDtypeStruct((_E, t), jnp.float32),
            jax.ShapeDtypeStruct((_E, 1, _D), jnp.float32),
        ],
    )(hs, router_w)

    col = scores[0, :_E]
    mask = col != 0.0
    nfir = jnp.sum(mask.astype(jnp.int32)).reshape(1)
    order = jnp.argsort(jnp.logical_not(mask), stable=True).astype(jnp.int32)
    slots = order[jnp.clip(jnp.arange(_E), 0, jnp.maximum(nfir[0] - 1, 0))]

    v8 = pl.pallas_call(
        _experts_body,
        grid_spec=pltpu.PrefetchScalarGridSpec(
            num_scalar_prefetch=2,
            grid=(_E, _NCK),
            in_specs=[
                pl.BlockSpec((1, 1, _D), _xs_im),
                pl.BlockSpec((1, _CF, _D), _wg_im),
                pl.BlockSpec((1, _CF, _D), _wg_im),
                pl.BlockSpec((1, _D, _CF), _wd_im),
            ],
            out_specs=pl.BlockSpec((_E, _D), lambda k, c, slots, nfir: (0, 0)),
        ),
        out_shape=jax.ShapeDtypeStruct((_E, _D), jnp.float32),
        compiler_params=pltpu.CompilerParams(
            dimension_semantics=("arbitrary", "arbitrary")),
    )(slots, nfir, xs, exp_wg, exp_wu, exp_wd)

    out = pl.pallas_call(
        _shared_body,
        grid=(t // _TMB, _NCK),
        in_specs=[
            pl.BlockSpec((_TMB, _D), lambda i, f: (i, 0)),
            pl.BlockSpec((_CF, _D), lambda i, f: (f, 0)),
            pl.BlockSpec((_CF, _D), lambda i, f: (f, 0)),
            pl.BlockSpec((_D, _CF), lambda i, f: (0, f)),
            pl.BlockSpec((_E, _D), lambda i, f: (0, 0)),
        ],
        out_specs=pl.BlockSpec((_TMB, _D), lambda i, f: (i, 0)),
        out_shape=jax.ShapeDtypeStruct((t, _D), jnp.float32),
        compiler_params=pltpu.CompilerParams(
            dimension_semantics=("parallel", "arbitrary")),
    )(hs, shared_wg, shared_wu, shared_wd, v8)

    return out, scores
